# Initial kernel scaffold; baseline (speedup 1.0000x reference)
import jax
import jax.numpy as jnp
from jax.experimental import pallas as pl


def kernel(atom_features, bond_features, bond_pairs, W1, b1, W2, b2, W3, b3):
    raise NotImplementedError("write your pallas kernel here")


# speedup vs baseline: 5.2889x; 5.2889x over previous
import functools

import jax
import jax.numpy as jnp
from jax import lax
from jax.experimental import pallas as pl
from jax.experimental.pallas import tpu as pltpu
from jax.experimental.pallas import tpu_sc as plsc

N = 10000
NPAD = 10240
E = 320000
EPAD = 2560 * 128
RP = 2560
DF = 128
U = 16
NC, NS, L = 2, 16, 16
NW = NC * NS
CR = 8
SCALE = (1.0 + 1e-3) ** -0.5


def _h_body(af_ref, w_ref, b_ref, o_ref):
    o_ref[...] = (
        jnp.dot(af_ref[...], w_ref[...], preferred_element_type=jnp.float32)
        + b_ref[...]
    )


def _h_call(af, w1s, b1s):
    return pl.pallas_call(
        _h_body,
        grid=(10,),
        in_specs=[
            pl.BlockSpec((1000, DF), lambda i: (i, 0)),
            pl.BlockSpec((DF, U), lambda i: (0, 0)),
            pl.BlockSpec((1, U), lambda i: (0, 0)),
        ],
        out_specs=pl.BlockSpec((1000, U), lambda i: (i, 0)),
        out_shape=jax.ShapeDtypeStruct((N, U), jnp.float32),
    )(af, w1s, b1s)


def _edge_body(bond_ref, nbrh_ref, w2bd_ref, b2_ref, s3_ref, b3_ref, p_ref,
               bf_ref, att8_ref):
    bf = (
        jnp.dot(bond_ref[...], w2bd_ref[...], preferred_element_type=jnp.float32)
        + b2_ref[...]
    )
    c = nbrh_ref[...] * bf
    apre = (
        jnp.dot(c, s3_ref[...], preferred_element_type=jnp.float32)
        + b3_ref[...]
    )
    a = jnp.exp(jnp.clip(apre, -2.0, 2.0))
    bf_ref[...] = bf
    att8_ref[...] = jnp.dot(a, p_ref[...], preferred_element_type=jnp.float32)


def _edge_call(bond2, nbrh2, w2bd, b2rep, s3, b3rep, p):
    return pl.pallas_call(
        _edge_body,
        grid=(40,),
        in_specs=[
            pl.BlockSpec((1000, 128), lambda i: (i, 0)),
            pl.BlockSpec((1000, 128), lambda i: (i, 0)),
            pl.BlockSpec((128, 128), lambda i: (0, 0)),
            pl.BlockSpec((1, 128), lambda i: (0, 0)),
            pl.BlockSpec((128, 128), lambda i: (0, 0)),
            pl.BlockSpec((1, 128), lambda i: (0, 0)),
            pl.BlockSpec((128, 8), lambda i: (0, 0)),
        ],
        out_specs=[
            pl.BlockSpec((1000, 128), lambda i: (i, 0)),
            pl.BlockSpec((1000, 8), lambda i: (i, 0)),
        ],
        out_shape=[
            jax.ShapeDtypeStruct((E // 8, 128), jnp.float32),
            jax.ShapeDtypeStruct((E // 8, 8), jnp.float32),
        ],
    )(bond2, nbrh2, w2bd, b2rep, s3, b3rep, p)


def _fin_body(bf_ref, nbrh_ref, attn8_ref, r_ref, out2_ref, w_ref):
    rep = jnp.dot(attn8_ref[...], r_ref[...], preferred_element_type=jnp.float32)
    out2_ref[...] = bf_ref[...] * rep
    w_ref[...] = nbrh_ref[...] * rep


def _fin_call(bf2, nbrh2, attn8, rm):
    return pl.pallas_call(
        _fin_body,
        grid=(40,),
        in_specs=[
            pl.BlockSpec((1000, 128), lambda i: (i, 0)),
            pl.BlockSpec((1000, 128), lambda i: (i, 0)),
            pl.BlockSpec((1000, 8), lambda i: (i, 0)),
            pl.BlockSpec((8, 128), lambda i: (0, 0)),
        ],
        out_specs=[
            pl.BlockSpec((1000, 128), lambda i: (i, 0)),
            pl.BlockSpec((1000, 128), lambda i: (i, 0)),
        ],
        out_shape=[
            jax.ShapeDtypeStruct((E // 8, 128), jnp.float32),
            jax.ShapeDtypeStruct((E // 8, 128), jnp.float32),
        ],
    )(bf2, nbrh2, attn8, rm)


def _comb_body(p_ref, o_ref):
    o_ref[...] = p_ref[0] + p_ref[1]


def _comb_call(partials):
    return pl.pallas_call(
        _comb_body,
        grid=(10,),
        in_specs=[pl.BlockSpec((2, 1000, U), lambda i: (0, i, 0))],
        out_specs=pl.BlockSpec((1000, U), lambda i: (i, 0)),
        out_shape=jax.ShapeDtypeStruct((N, U), jnp.float32),
    )(partials)


_MESH = plsc.VectorSubcoreMesh(core_axis_name="c", subcore_axis_name="s")


@functools.partial(
    pl.kernel,
    out_type=jax.ShapeDtypeStruct((EPAD, U), jnp.float32),
    mesh=_MESH,
    compiler_params=pltpu.CompilerParams(needs_layout_passes=False, use_tc_tiling_on_sc=False),
    scratch_types=[
        pltpu.VMEM((CR, 128), jnp.int32),
        pltpu.VMEM((CR * 128, U), jnp.float32),
        pltpu.VMEM_SHARED((N, U), jnp.float32),
    ],
)
def _gather_k(h_hbm, dst_hbm, out_hbm, idx_v, rows_v, h_sh):
    c = lax.axis_index("c")
    s = lax.axis_index("s")
    w = s * NC + c

    @pl.when(s == 0)
    def _():
        pltpu.sync_copy(h_hbm, h_sh)

    plsc.subcore_barrier()
    r0 = w * (RP // NW)

    def chunk(ci, _):
        rbase = r0 + ci * CR
        pltpu.sync_copy(dst_hbm.at[pl.ds(rbase, CR)], idx_v)
        for j in range(CR):
            pltpu.sync_copy(
                h_sh.at[idx_v.at[j]], rows_v.at[pl.ds(j * 128, 128)]
            )
        pltpu.sync_copy(rows_v, out_hbm.at[pl.ds(rbase * 128, CR * 128)])
        return None

    lax.fori_loop(0, (RP // NW) // CR, chunk, None)


@functools.partial(
    pl.kernel,
    out_type=jax.ShapeDtypeStruct((E,), jnp.float32),
    mesh=_MESH,
    compiler_params=pltpu.CompilerParams(needs_layout_passes=False, use_tc_tiling_on_sc=False),
    scratch_types=[
        pltpu.VMEM_SHARED((NPAD,), jnp.float32),
        pltpu.VMEM_SHARED((NPAD,), jnp.int32),
        pltpu.VMEM((CR, 128), jnp.int32),
        pltpu.VMEM((CR, 128), jnp.float32),
        pltpu.VMEM((128,), jnp.int32),
        pltpu.VMEM((N,), jnp.float32),
        pltpu.VMEM((N,), jnp.int32),
        pltpu.VMEM((N,), jnp.int32),
        pltpu.VMEM((79 * 128,), jnp.int32),
        pltpu.VMEM((79 * 128,), jnp.float32),
        pltpu.VMEM((79 * 128,), jnp.float32),
    ],
)
def _norm_k(att1_hbm, att2_hbm, src2_hbm, zf_hbm, zi_hbm, out_hbm,
            sums_sh, cnts_sh, idx_v, att_st, ones_v,
            sums_v, cnts_v, off_v, z_v, attb_v, attn_v):
    c = lax.axis_index("c")
    s = lax.axis_index("s")
    w = s * NC + c

    @pl.when(s == 0)
    def _():
        pltpu.sync_copy(zf_hbm, sums_sh)
        pltpu.sync_copy(zi_hbm, cnts_sh)

    for j in range(8):
        ones_v[pl.ds(j * 16, 16)] = jnp.ones((16,), jnp.int32)

    plsc.subcore_barrier()

    ra0 = s * (RP // NS)

    def chunk_a(ci, _):
        rbase = ra0 + ci * CR
        pltpu.sync_copy(src2_hbm.at[pl.ds(rbase, CR)], idx_v)
        pltpu.sync_copy(att2_hbm.at[pl.ds(rbase, CR)], att_st)
        for j in range(CR):
            pltpu.sync_copy(att_st.at[j], sums_sh.at[idx_v.at[j]], add=True)
            pltpu.sync_copy(ones_v, cnts_sh.at[idx_v.at[j]], add=True)
        return None

    lax.fori_loop(0, (RP // NS) // CR, chunk_a, None)

    plsc.subcore_barrier()

    pltpu.sync_copy(sums_sh.at[pl.ds(0, N)], sums_v)
    pltpu.sync_copy(cnts_sh.at[pl.ds(0, N)], cnts_v)

    rb0 = 78 * w + jnp.minimum(w, 4)
    a0 = rb0 * 128
    nrb = 78 + jnp.where(w < 4, 1, 0)
    npos = nrb * 128
    nv = nrb * 8

    def scan_body(j, carry):
        tot, acc = carry
        ds = pl.ds(j * 16, 16)
        cv = cnts_v[ds]
        incl = plsc.cumsum(cv) + tot
        offv = incl - cv
        off_v[ds] = offv
        acc = acc + plsc.all_reduce_population_count(offv <= a0)
        return (jnp.max(incl), acc)

    _, acc = lax.fori_loop(
        0, N // 16, scan_body, (jnp.int32(0), jnp.zeros((16,), jnp.int32))
    )
    p_a = jnp.max(acc) - 1

    @pl.when(w < 4)
    def _():
        pltpu.sync_copy(att1_hbm.at[pl.ds(a0, 79 * 128)], attb_v)

    @pl.when(w >= 4)
    def _():
        pltpu.sync_copy(
            att1_hbm.at[pl.ds(a0, 78 * 128)], attb_v.at[pl.ds(0, 78 * 128)]
        )

    def init_z(j, _):
        z_v[pl.ds(j * 16, 16)] = jnp.zeros((16,), jnp.int32) + p_a
        return None

    lax.fori_loop(0, nv, init_z, None)

    iota = lax.iota(jnp.int32, 16)

    def scat(j, _):
        ds = pl.ds(j * 16, 16)
        offv = off_v[ds]
        cv = cnts_v[ds]
        ids = iota + j * 16
        m = (cv > 0) & (offv >= a0) & (offv < a0 + npos)
        plsc.store_scatter(z_v, [offv - a0], ids, mask=m)
        return None

    lax.fori_loop(0, N // 16, scat, None)

    def fill(j, carry):
        ds = pl.ds(j * 16, 16)
        zc = jnp.maximum(plsc.cummax(z_v[ds]), carry)
        sv = plsc.load_gather(sums_v, [zc])
        attn_v[ds] = attb_v[ds] / sv
        return jnp.max(zc)

    lax.fori_loop(0, nv, fill, p_a)

    @pl.when(w < 4)
    def _():
        pltpu.sync_copy(attn_v, out_hbm.at[pl.ds(a0, 79 * 128)])

    @pl.when(w >= 4)
    def _():
        pltpu.sync_copy(
            attn_v.at[pl.ds(0, 78 * 128)], out_hbm.at[pl.ds(a0, 78 * 128)]
        )


@functools.partial(
    pl.kernel,
    out_type=jax.ShapeDtypeStruct((NC, N, U), jnp.float32),
    mesh=_MESH,
    compiler_params=pltpu.CompilerParams(needs_layout_passes=False, use_tc_tiling_on_sc=False),
    scratch_types=[
        pltpu.VMEM_SHARED((NPAD, U), jnp.float32),
        pltpu.VMEM((CR, 128), jnp.int32),
        pltpu.VMEM((CR * 128, U), jnp.float32),
    ],
)
def _scat_k(w_hbm, src2_hbm, z16_hbm, out_hbm, acc_sh, idx_v, w_v):
    c = lax.axis_index("c")
    s = lax.axis_index("s")

    @pl.when(s == 0)
    def _():
        pltpu.sync_copy(z16_hbm, acc_sh)

    plsc.subcore_barrier()

    r0 = c * (RP // NC) + s * (RP // NC // NS)

    def chunk(ci, _):
        rbase = r0 + ci * CR
        pltpu.sync_copy(src2_hbm.at[pl.ds(rbase, CR)], idx_v)
        pltpu.sync_copy(w_hbm.at[pl.ds(rbase * 128, CR * 128)], w_v)
        for j in range(CR):
            pltpu.sync_copy(
                w_v.at[pl.ds(j * 128, 128)], acc_sh.at[idx_v.at[j]], add=True
            )
        return None

    lax.fori_loop(0, (RP // NC // NS) // CR, chunk, None)

    plsc.subcore_barrier()
    pltpu.sync_copy(
        acc_sh.at[pl.ds(s * (N // NS), N // NS)],
        out_hbm.at[c, pl.ds(s * (N // NS), N // NS)],
    )


def kernel(atom_features, bond_features, bond_pairs, W1, b1, W2, b2, W3, b3):
    f32 = jnp.float32
    i32 = jnp.int32
    w1s = W1 * SCALE
    b1s = (b1 * SCALE).reshape(1, U)
    w2s = W2 * SCALE
    b2s = b2 * SCALE
    w3s = W3 * SCALE
    b3s = b3 * SCALE

    eye8 = jnp.eye(8, dtype=f32)
    w2bd = jnp.kron(eye8, w2s)
    s3 = jnp.kron(eye8, w3s @ jnp.ones((1, U), f32))
    p = jnp.kron(eye8, jnp.eye(U, 1, dtype=f32))
    rm = jnp.kron(eye8, jnp.ones((1, U), f32))
    b2rep = jnp.tile(b2s, 8).reshape(1, 128)
    b3rep = jnp.full((1, 128), b3s[0], f32)

    npad = EPAD - E
    src = bond_pairs[:, 0]
    dst = bond_pairs[:, 1]
    src2p = jnp.concatenate([src, jnp.full((npad,), N, i32)]).reshape(RP, 128)
    dst2p = jnp.concatenate([dst, jnp.zeros((npad,), i32)]).reshape(RP, 128)

    h = _h_call(atom_features, w1s, b1s)
    nbrh = _gather_k(h, dst2p)

    bond2 = bond_features.reshape(E // 8, 128)
    nbrh2 = nbrh.reshape(EPAD // 8, 128)
    bf2, att8 = _edge_call(bond2, nbrh2, w2bd, b2rep, s3, b3rep, p)
    att1 = att8.reshape(E)
    att2p = jnp.concatenate([att1, jnp.zeros((npad,), f32)]).reshape(RP, 128)

    zf = jnp.zeros((NPAD,), f32)
    zi = jnp.zeros((NPAD,), i32)
    attn = _norm_k(att1, att2p, src2p, zf, zi)

    out2_2, w2m = _fin_call(bf2, nbrh2, attn.reshape(E // 8, 8), rm)

    wpad = jnp.concatenate([w2m.reshape(E, U), jnp.zeros((npad, U), f32)])
    z16 = jnp.zeros((NPAD, U), f32)
    partials = _scat_k(wpad, src2p, z16)
    out1 = _comb_call(partials)
    return (out1, out2_2.reshape(E, U))


# speedup vs baseline: 9.1331x; 1.7268x over previous
import functools

import jax
import jax.numpy as jnp
from jax import lax
from jax.experimental import pallas as pl
from jax.experimental.pallas import tpu as pltpu
from jax.experimental.pallas import tpu_sc as plsc

N = 10000
NPAD = 10240
E = 320000
DF = 128
U = 16
NC, NS, L = 2, 16, 16
NW = NC * NS
CH = 2000
EW = E // NW
ET = E // NC // NS
SCALE = (1.0 + 1e-3) ** -0.5


def _h_body(af_ref, w_ref, b_ref, o_ref):
    o_ref[...] = (
        jnp.dot(af_ref[...], w_ref[...], preferred_element_type=jnp.float32)
        + b_ref[...]
    )


def _h_call(af, w1s, b1s):
    return pl.pallas_call(
        _h_body,
        grid=(10,),
        in_specs=[
            pl.BlockSpec((1000, DF), lambda i: (i, 0)),
            pl.BlockSpec((DF, U), lambda i: (0, 0)),
            pl.BlockSpec((1, U), lambda i: (0, 0)),
        ],
        out_specs=pl.BlockSpec((1000, U), lambda i: (i, 0)),
        out_shape=jax.ShapeDtypeStruct((N, U), jnp.float32),
    )(af, w1s, b1s)


def _edge_body(bond_ref, nbrh_ref, w2bd_ref, b2_ref, s3_ref, b3_ref, p_ref,
               att8_ref):
    bf = (
        jnp.dot(bond_ref[...], w2bd_ref[...], preferred_element_type=jnp.float32)
        + b2_ref[...]
    )
    c = nbrh_ref[...] * bf
    apre = (
        jnp.dot(c, s3_ref[...], preferred_element_type=jnp.float32)
        + b3_ref[...]
    )
    a = jnp.exp(jnp.clip(apre, -2.0, 2.0))
    att8_ref[...] = jnp.dot(a, p_ref[...], preferred_element_type=jnp.float32)


def _edge_call(bond2, nbrh2, w2bd, b2rep, s3, b3rep, p):
    return pl.pallas_call(
        _edge_body,
        grid=(40,),
        in_specs=[
            pl.BlockSpec((1000, 128), lambda i: (i, 0)),
            pl.BlockSpec((1000, 128), lambda i: (i, 0)),
            pl.BlockSpec((128, 128), lambda i: (0, 0)),
            pl.BlockSpec((1, 128), lambda i: (0, 0)),
            pl.BlockSpec((128, 128), lambda i: (0, 0)),
            pl.BlockSpec((1, 128), lambda i: (0, 0)),
            pl.BlockSpec((128, 8), lambda i: (0, 0)),
        ],
        out_specs=pl.BlockSpec((1000, 8), lambda i: (i, 0)),
        out_shape=jax.ShapeDtypeStruct((E // 8, 8), jnp.float32),
    )(bond2, nbrh2, w2bd, b2rep, s3, b3rep, p)


def _fin_body(bond_ref, attn8_ref, w2bd_ref, b2_ref, r_ref, out2_ref):
    rep = jnp.dot(attn8_ref[...], r_ref[...], preferred_element_type=jnp.float32)
    bf = (
        jnp.dot(bond_ref[...], w2bd_ref[...], preferred_element_type=jnp.float32)
        + b2_ref[...]
    )
    out2_ref[...] = bf * rep


def _fin_call(bond2, attn8, w2bd, b2rep, rm):
    return pl.pallas_call(
        _fin_body,
        grid=(40,),
        in_specs=[
            pl.BlockSpec((1000, 128), lambda i: (i, 0)),
            pl.BlockSpec((1000, 8), lambda i: (i, 0)),
            pl.BlockSpec((128, 128), lambda i: (0, 0)),
            pl.BlockSpec((1, 128), lambda i: (0, 0)),
            pl.BlockSpec((8, 128), lambda i: (0, 0)),
        ],
        out_specs=pl.BlockSpec((1000, 128), lambda i: (i, 0)),
        out_shape=jax.ShapeDtypeStruct((E // 8, 128), jnp.float32),
    )(bond2, attn8, w2bd, b2rep, rm)


def _comb_body(p_ref, o_ref):
    o_ref[...] = p_ref[0] + p_ref[1]


def _comb_call(partials):
    return pl.pallas_call(
        _comb_body,
        grid=(10,),
        in_specs=[pl.BlockSpec((2, 1000, U), lambda i: (0, i, 0))],
        out_specs=pl.BlockSpec((1000, U), lambda i: (i, 0)),
        out_shape=jax.ShapeDtypeStruct((N, U), jnp.float32),
    )(partials)


_MESH = plsc.VectorSubcoreMesh(core_axis_name="c", subcore_axis_name="s")
_SC_PARAMS = pltpu.CompilerParams(
    needs_layout_passes=False, use_tc_tiling_on_sc=False
)


@functools.partial(
    pl.kernel,
    out_type=jax.ShapeDtypeStruct((E, U), jnp.float32),
    mesh=_MESH,
    compiler_params=_SC_PARAMS,
    scratch_types=[
        pltpu.VMEM((CH,), jnp.int32),
        pltpu.VMEM((CH, U), jnp.float32),
        pltpu.VMEM_SHARED((N, U), jnp.float32),
        pltpu.SemaphoreType.DMA,
    ],
)
def _gather_k(h_hbm, dst_hbm, out_hbm, idx_v, rows_v, h_sh, sem):
    c = lax.axis_index("c")
    s = lax.axis_index("s")
    w = s * NC + c

    @pl.when(s == 0)
    def _():
        pltpu.sync_copy(h_hbm, h_sh)

    plsc.subcore_barrier()
    e0 = w * EW

    def chunk(ci, _):
        base = e0 + ci * CH
        pltpu.sync_copy(dst_hbm.at[pl.ds(base, CH)], idx_v)
        pltpu.async_copy(h_sh.at[idx_v], rows_v, sem).wait()
        pltpu.sync_copy(rows_v, out_hbm.at[pl.ds(base, CH)])
        return None

    lax.fori_loop(0, EW // CH, chunk, None)


@functools.partial(
    pl.kernel,
    out_type=jax.ShapeDtypeStruct((E,), jnp.float32),
    mesh=_MESH,
    compiler_params=_SC_PARAMS,
    scratch_types=[
        pltpu.VMEM_SHARED((NPAD,), jnp.float32),
        pltpu.VMEM_SHARED((NPAD,), jnp.int32),
        pltpu.VMEM((CH,), jnp.int32),
        pltpu.VMEM((CH,), jnp.float32),
        pltpu.VMEM((CH,), jnp.int32),
        pltpu.VMEM((N,), jnp.float32),
        pltpu.VMEM((N,), jnp.int32),
        pltpu.VMEM((N,), jnp.int32),
        pltpu.VMEM((EW,), jnp.int32),
        pltpu.VMEM((EW,), jnp.float32),
        pltpu.VMEM((EW,), jnp.float32),
        pltpu.SemaphoreType.DMA,
    ],
)
def _norm_k(att1_hbm, src_hbm, zf_hbm, zi_hbm, out_hbm,
            sums_sh, cnts_sh, idx_v, att_st, ones_v,
            sums_v, cnts_v, off_v, z_v, attb_v, attn_v, sem):
    c = lax.axis_index("c")
    s = lax.axis_index("s")
    w = s * NC + c

    @pl.when(s == 0)
    def _():
        pltpu.sync_copy(zf_hbm, sums_sh)
        pltpu.sync_copy(zi_hbm, cnts_sh)

    def fill_ones(j, _):
        ones_v[pl.ds(j * 16, 16)] = jnp.ones((16,), jnp.int32)
        return None

    lax.fori_loop(0, CH // 16, fill_ones, None)

    plsc.subcore_barrier()

    ea0 = s * (E // NS)

    def chunk_a(ci, _):
        base = ea0 + ci * CH
        pltpu.sync_copy(src_hbm.at[pl.ds(base, CH)], idx_v)
        pltpu.sync_copy(att1_hbm.at[pl.ds(base, CH)], att_st)
        d1 = pltpu.async_copy(att_st, sums_sh.at[idx_v], sem, add=True)
        d2 = pltpu.async_copy(ones_v, cnts_sh.at[idx_v], sem, add=True)
        d1.wait()
        d2.wait()
        return None

    lax.fori_loop(0, (E // NS) // CH, chunk_a, None)

    plsc.subcore_barrier()

    pltpu.sync_copy(sums_sh.at[pl.ds(0, N)], sums_v)
    pltpu.sync_copy(cnts_sh.at[pl.ds(0, N)], cnts_v)

    a0 = w * EW
    lane15 = jnp.zeros((16,), jnp.int32) + 15

    def scan_body(j, carry):
        tot, acc = carry
        ds = pl.ds(j * 16, 16)
        cv = cnts_v[ds]
        incl = plsc.cumsum(cv) + tot
        offv = incl - cv
        off_v[ds] = offv
        acc = acc + plsc.all_reduce_population_count(offv <= a0)
        return (incl.at[lane15].get(mode="promise_in_bounds"), acc)

    _, acc = lax.fori_loop(
        0, N // 16, scan_body,
        (jnp.zeros((16,), jnp.int32), jnp.zeros((16,), jnp.int32)),
    )
    p_a = jnp.max(acc) - 1

    pltpu.sync_copy(att1_hbm.at[pl.ds(a0, EW)], attb_v)

    def init_z(j, _):
        z_v[pl.ds(j * 16, 16)] = jnp.zeros((16,), jnp.int32) + p_a
        return None

    lax.fori_loop(0, EW // 16, init_z, None)

    iota = lax.iota(jnp.int32, 16)

    def scat(j, _):
        ds = pl.ds(j * 16, 16)
        offv = off_v[ds]
        cv = cnts_v[ds]
        ids = iota + j * 16
        m = (cv > 0) & (offv >= a0) & (offv < a0 + EW)
        plsc.store_scatter(z_v, [offv - a0], ids, mask=m)
        return None

    lax.fori_loop(0, N // 16, scat, None)

    def fill(j, carry):
        ds = pl.ds(j * 16, 16)
        zc = jnp.maximum(plsc.cummax(z_v[ds]), carry)
        sv = plsc.load_gather(sums_v, [zc])
        attn_v[ds] = attb_v[ds] / sv
        return zc.at[lane15].get(mode="promise_in_bounds")

    lax.fori_loop(0, EW // 16, fill, jnp.zeros((16,), jnp.int32) + p_a)

    pltpu.sync_copy(attn_v, out_hbm.at[pl.ds(a0, EW)])


@functools.partial(
    pl.kernel,
    out_type=jax.ShapeDtypeStruct((NC, N, U), jnp.float32),
    mesh=_MESH,
    compiler_params=_SC_PARAMS,
    scratch_types=[
        pltpu.VMEM_SHARED((NPAD, U), jnp.float32),
        pltpu.VMEM((CH,), jnp.int32),
        pltpu.VMEM((CH, U), jnp.float32),
        pltpu.VMEM((CH,), jnp.float32),
        pltpu.VMEM((CH, U), jnp.float32),
        pltpu.SemaphoreType.DMA,
    ],
)
def _scat_k(nbrh_hbm, attn_hbm, src_hbm, z16_hbm, out_hbm,
            acc_sh, idx_v, nb_v, at_v, w_v, sem):
    c = lax.axis_index("c")
    s = lax.axis_index("s")

    @pl.when(s == 0)
    def _():
        pltpu.sync_copy(z16_hbm, acc_sh)

    plsc.subcore_barrier()

    e0 = c * (E // NC) + s * ET

    def chunk(ci, _):
        base = e0 + ci * CH
        pltpu.sync_copy(src_hbm.at[pl.ds(base, CH)], idx_v)
        pltpu.sync_copy(nbrh_hbm.at[pl.ds(base, CH)], nb_v)
        pltpu.sync_copy(attn_hbm.at[pl.ds(base, CH)], at_v)

        def rows(m, _):
            av = at_v[pl.ds(m * 16, 16)]
            for k in range(16):
                scale = av.at[jnp.zeros((16,), jnp.int32) + k].get(
                    mode="promise_in_bounds")
                w_v[m * 16 + k] = nb_v[m * 16 + k] * scale
            return None

        lax.fori_loop(0, CH // 16, rows, None)
        pltpu.async_copy(w_v, acc_sh.at[idx_v], sem, add=True).wait()
        return None

    lax.fori_loop(0, ET // CH, chunk, None)

    plsc.subcore_barrier()
    pltpu.sync_copy(
        acc_sh.at[pl.ds(s * (N // NS), N // NS)],
        out_hbm.at[c, pl.ds(s * (N // NS), N // NS)],
    )


def kernel(atom_features, bond_features, bond_pairs, W1, b1, W2, b2, W3, b3):
    f32 = jnp.float32
    i32 = jnp.int32
    w1s = W1 * SCALE
    b1s = (b1 * SCALE).reshape(1, U)
    w2s = W2 * SCALE
    b2s = b2 * SCALE
    w3s = W3 * SCALE
    b3s = b3 * SCALE

    eye8 = jnp.eye(8, dtype=f32)
    w2bd = jnp.kron(eye8, w2s)
    s3 = jnp.kron(eye8, w3s @ jnp.ones((1, U), f32))
    p = jnp.kron(eye8, jnp.eye(U, 1, dtype=f32))
    rm = jnp.kron(eye8, jnp.ones((1, U), f32))
    b2rep = jnp.tile(b2s, 8).reshape(1, 128)
    b3rep = jnp.full((1, 128), b3s[0], f32)

    src = bond_pairs[:, 0]
    dst = bond_pairs[:, 1]

    h = _h_call(atom_features, w1s, b1s)
    nbrh = _gather_k(h, dst)

    bond2 = bond_features.reshape(E // 8, 128)
    nbrh2 = nbrh.reshape(E // 8, 128)
    att8 = _edge_call(bond2, nbrh2, w2bd, b2rep, s3, b3rep, p)
    att1 = att8.reshape(E)

    zf = jnp.zeros((NPAD,), f32)
    zi = jnp.zeros((NPAD,), i32)
    attn = _norm_k(att1, src, zf, zi)

    out2w = _fin_call(bond2, attn.reshape(E // 8, 8), w2bd, b2rep, rm)

    z16 = jnp.zeros((NPAD, U), f32)
    partials = _scat_k(nbrh, attn, src, z16)
    out1 = _comb_call(partials)
    return (out1, out2w.reshape(E, U))


# speedup vs baseline: 9.7381x; 1.0662x over previous
import functools

import jax
import jax.numpy as jnp
from jax import lax
from jax.experimental import pallas as pl
from jax.experimental.pallas import tpu as pltpu
from jax.experimental.pallas import tpu_sc as plsc

N = 10000
NPAD = 10240
E = 320000
DF = 128
U = 16
NC, NS, L = 2, 16, 16
NW = NC * NS
CH = 2000
EW = E // NW
ET = E // NC // NS
SCALE = (1.0 + 1e-3) ** -0.5


def _h_body(af_ref, w_ref, b_ref, o_ref):
    o_ref[...] = (
        jnp.dot(af_ref[...], w_ref[...], preferred_element_type=jnp.float32)
        + b_ref[...]
    )


def _h_call(af, w1s, b1s):
    return pl.pallas_call(
        _h_body,
        grid=(10,),
        in_specs=[
            pl.BlockSpec((1000, DF), lambda i: (i, 0)),
            pl.BlockSpec((DF, U), lambda i: (0, 0)),
            pl.BlockSpec((1, U), lambda i: (0, 0)),
        ],
        out_specs=pl.BlockSpec((1000, U), lambda i: (i, 0)),
        out_shape=jax.ShapeDtypeStruct((N, U), jnp.float32),
    )(af, w1s, b1s)


def _edge_body(bond_ref, nbrh_ref, w2bd_ref, b2_ref, s3_ref, b3_ref, p_ref,
               att8_ref, bf_ref):
    bf = (
        jnp.dot(bond_ref[...], w2bd_ref[...], preferred_element_type=jnp.float32)
        + b2_ref[...]
    )
    c = nbrh_ref[...] * bf
    apre = (
        jnp.dot(c, s3_ref[...], preferred_element_type=jnp.float32)
        + b3_ref[...]
    )
    a = jnp.exp(jnp.clip(apre, -2.0, 2.0))
    att8_ref[...] = jnp.dot(a, p_ref[...], preferred_element_type=jnp.float32)
    bf_ref[...] = bf


def _edge_call(bond2, nbrh2, w2bd, b2rep, s3, b3rep, p):
    return pl.pallas_call(
        _edge_body,
        grid=(40,),
        in_specs=[
            pl.BlockSpec((1000, 128), lambda i: (i, 0)),
            pl.BlockSpec((1000, 128), lambda i: (i, 0)),
            pl.BlockSpec((128, 128), lambda i: (0, 0)),
            pl.BlockSpec((1, 128), lambda i: (0, 0)),
            pl.BlockSpec((128, 128), lambda i: (0, 0)),
            pl.BlockSpec((1, 128), lambda i: (0, 0)),
            pl.BlockSpec((128, 8), lambda i: (0, 0)),
        ],
        out_specs=[
            pl.BlockSpec((1000, 8), lambda i: (i, 0)),
            pl.BlockSpec((1000, 128), lambda i: (i, 0)),
        ],
        out_shape=[
            jax.ShapeDtypeStruct((E // 8, 8), jnp.float32),
            jax.ShapeDtypeStruct((E // 8, 128), jnp.float32),
        ],
    )(bond2, nbrh2, w2bd, b2rep, s3, b3rep, p)


def _comb_body(p_ref, o_ref):
    o_ref[...] = p_ref[0] + p_ref[1]


def _comb_call(partials):
    return pl.pallas_call(
        _comb_body,
        grid=(10,),
        in_specs=[pl.BlockSpec((2, 1000, U), lambda i: (0, i, 0))],
        out_specs=pl.BlockSpec((1000, U), lambda i: (i, 0)),
        out_shape=jax.ShapeDtypeStruct((N, U), jnp.float32),
    )(partials)


_MESH = plsc.VectorSubcoreMesh(core_axis_name="c", subcore_axis_name="s")
_SC_PARAMS = pltpu.CompilerParams(
    needs_layout_passes=False, use_tc_tiling_on_sc=False
)


@functools.partial(
    pl.kernel,
    out_type=jax.ShapeDtypeStruct((E, U), jnp.float32),
    mesh=_MESH,
    compiler_params=_SC_PARAMS,
    scratch_types=[
        pltpu.VMEM((CH,), jnp.int32),
        pltpu.VMEM((CH, U), jnp.float32),
        pltpu.VMEM_SHARED((N, U), jnp.float32),
        pltpu.SemaphoreType.DMA,
    ],
)
def _gather_k(h_hbm, dst_hbm, out_hbm, idx_v, rows_v, h_sh, sem):
    c = lax.axis_index("c")
    s = lax.axis_index("s")
    w = s * NC + c

    @pl.when(s == 0)
    def _():
        pltpu.sync_copy(h_hbm, h_sh)

    plsc.subcore_barrier()
    e0 = w * EW

    def chunk(ci, _):
        base = e0 + ci * CH
        pltpu.sync_copy(dst_hbm.at[pl.ds(base, CH)], idx_v)
        pltpu.async_copy(h_sh.at[idx_v], rows_v, sem).wait()
        pltpu.sync_copy(rows_v, out_hbm.at[pl.ds(base, CH)])
        return None

    lax.fori_loop(0, EW // CH, chunk, None)


@functools.partial(
    pl.kernel,
    out_type=jax.ShapeDtypeStruct((E,), jnp.float32),
    mesh=_MESH,
    compiler_params=_SC_PARAMS,
    scratch_types=[
        pltpu.VMEM_SHARED((NPAD,), jnp.float32),
        pltpu.VMEM_SHARED((NPAD,), jnp.int32),
        pltpu.VMEM((CH,), jnp.int32),
        pltpu.VMEM((CH,), jnp.float32),
        pltpu.VMEM((CH,), jnp.int32),
        pltpu.VMEM((N,), jnp.float32),
        pltpu.VMEM((N,), jnp.int32),
        pltpu.VMEM((N,), jnp.int32),
        pltpu.VMEM((EW,), jnp.int32),
        pltpu.VMEM((EW,), jnp.float32),
        pltpu.VMEM((EW,), jnp.float32),
        pltpu.SemaphoreType.DMA,
    ],
)
def _norm_k(att1_hbm, src_hbm, zf_hbm, zi_hbm, out_hbm,
            sums_sh, cnts_sh, idx_v, att_st, ones_v,
            sums_v, cnts_v, off_v, z_v, attb_v, attn_v, sem):
    c = lax.axis_index("c")
    s = lax.axis_index("s")
    w = s * NC + c

    @pl.when(s == 0)
    def _():
        pltpu.sync_copy(zf_hbm, sums_sh)
        pltpu.sync_copy(zi_hbm, cnts_sh)

    def fill_ones(j, _):
        ones_v[pl.ds(j * 16, 16)] = jnp.ones((16,), jnp.int32)
        return None

    lax.fori_loop(0, CH // 16, fill_ones, None)

    plsc.subcore_barrier()

    ea0 = s * (E // NS)

    def chunk_a(ci, _):
        base = ea0 + ci * CH
        pltpu.sync_copy(src_hbm.at[pl.ds(base, CH)], idx_v)
        pltpu.sync_copy(att1_hbm.at[pl.ds(base, CH)], att_st)
        d1 = pltpu.async_copy(att_st, sums_sh.at[idx_v], sem, add=True)
        d2 = pltpu.async_copy(ones_v, cnts_sh.at[idx_v], sem, add=True)
        d1.wait()
        d2.wait()
        return None

    lax.fori_loop(0, (E // NS) // CH, chunk_a, None)

    plsc.subcore_barrier()

    pltpu.sync_copy(sums_sh.at[pl.ds(0, N)], sums_v)
    pltpu.sync_copy(cnts_sh.at[pl.ds(0, N)], cnts_v)

    a0 = w * EW
    lane15 = jnp.zeros((16,), jnp.int32) + 15

    def scan_body(j, carry):
        tot, acc = carry
        ds = pl.ds(j * 16, 16)
        cv = cnts_v[ds]
        incl = plsc.cumsum(cv) + tot
        offv = incl - cv
        off_v[ds] = offv
        acc = acc + plsc.all_reduce_population_count(offv <= a0)
        return (incl.at[lane15].get(mode="promise_in_bounds"), acc)

    _, acc = lax.fori_loop(
        0, N // 16, scan_body,
        (jnp.zeros((16,), jnp.int32), jnp.zeros((16,), jnp.int32)),
    )
    p_a = jnp.max(acc) - 1

    pltpu.sync_copy(att1_hbm.at[pl.ds(a0, EW)], attb_v)

    def init_z(j, _):
        z_v[pl.ds(j * 16, 16)] = jnp.zeros((16,), jnp.int32) + p_a
        return None

    lax.fori_loop(0, EW // 16, init_z, None)

    iota = lax.iota(jnp.int32, 16)

    def scat(j, _):
        ds = pl.ds(j * 16, 16)
        offv = off_v[ds]
        cv = cnts_v[ds]
        ids = iota + j * 16
        m = (cv > 0) & (offv >= a0) & (offv < a0 + EW)
        plsc.store_scatter(z_v, [offv - a0], ids, mask=m)
        return None

    lax.fori_loop(0, N // 16, scat, None)

    def fill(j, carry):
        ds = pl.ds(j * 16, 16)
        zc = jnp.maximum(plsc.cummax(z_v[ds]), carry)
        sv = plsc.load_gather(sums_v, [zc])
        attn_v[ds] = attb_v[ds] / sv
        return zc.at[lane15].get(mode="promise_in_bounds")

    lax.fori_loop(0, EW // 16, fill, jnp.zeros((16,), jnp.int32) + p_a)

    pltpu.sync_copy(attn_v, out_hbm.at[pl.ds(a0, EW)])


@functools.partial(
    pl.kernel,
    out_type=[
        jax.ShapeDtypeStruct((NC, N, U), jnp.float32),
        jax.ShapeDtypeStruct((U, E), jnp.float32),
    ],
    mesh=_MESH,
    compiler_params=_SC_PARAMS,
    scratch_types=[
        pltpu.VMEM_SHARED((NPAD, U), jnp.float32),
        pltpu.VMEM((CH,), jnp.int32),
        pltpu.VMEM((CH, U), jnp.float32),
        pltpu.VMEM((CH, U), jnp.float32),
        pltpu.VMEM((CH,), jnp.float32),
        pltpu.VMEM((U * CH,), jnp.float32),
        pltpu.SemaphoreType.DMA,
        pltpu.SemaphoreType.DMA,
    ],
)
def _scat_k(nbrh_hbm, bf_hbm, attn_hbm, src_hbm, z16_hbm, out_hbm, out2t_hbm,
            acc_sh, idx_v, nb_v, bfv, at_v, o2t_v, sem, sem2):
    c = lax.axis_index("c")
    s = lax.axis_index("s")

    @pl.when(s == 0)
    def _():
        pltpu.sync_copy(z16_hbm, acc_sh)

    plsc.subcore_barrier()

    e0 = c * (E // NC) + s * ET

    def chunk(ci, _):
        base = e0 + ci * CH
        pltpu.sync_copy(src_hbm.at[pl.ds(base, CH)], idx_v)
        pltpu.sync_copy(nbrh_hbm.at[pl.ds(base, CH)], nb_v)
        pltpu.sync_copy(bf_hbm.at[pl.ds(base, CH)], bfv)
        pltpu.sync_copy(attn_hbm.at[pl.ds(base, CH)], at_v)

        colbase = lax.iota(jnp.int32, 16) * CH

        def rows(m, _):
            av = at_v[pl.ds(m * 16, 16)]
            for k in range(16):
                e = m * 16 + k
                scale = av.at[jnp.zeros((16,), jnp.int32) + k].get(
                    mode="promise_in_bounds")
                nb_v[e] = nb_v[e] * scale
                plsc.store_scatter(
                    o2t_v, [colbase + e], bfv[e] * scale
                )
            return None

        lax.fori_loop(0, CH // 16, rows, None)
        d1 = pltpu.async_copy(nb_v, acc_sh.at[idx_v], sem, add=True)
        descs = []
        for k in range(U):
            descs.append(pltpu.async_copy(
                o2t_v.at[pl.ds(k * CH, CH)],
                out2t_hbm.at[k, pl.ds(base, CH)],
                sem2,
            ))
        d1.wait()
        for d in descs:
            d.wait()
        return None

    lax.fori_loop(0, ET // CH, chunk, None)

    plsc.subcore_barrier()
    pltpu.sync_copy(
        acc_sh.at[pl.ds(s * (N // NS), N // NS)],
        out_hbm.at[c, pl.ds(s * (N // NS), N // NS)],
    )


def kernel(atom_features, bond_features, bond_pairs, W1, b1, W2, b2, W3, b3):
    f32 = jnp.float32
    i32 = jnp.int32
    w1s = W1 * SCALE
    b1s = (b1 * SCALE).reshape(1, U)
    w2s = W2 * SCALE
    b2s = b2 * SCALE
    w3s = W3 * SCALE
    b3s = b3 * SCALE

    eye8 = jnp.eye(8, dtype=f32)
    w2bd = jnp.kron(eye8, w2s)
    s3 = jnp.kron(eye8, w3s @ jnp.ones((1, U), f32))
    p = jnp.kron(eye8, jnp.eye(U, 1, dtype=f32))
    b2rep = jnp.tile(b2s, 8).reshape(1, 128)
    b3rep = jnp.full((1, 128), b3s[0], f32)

    pt = bond_pairs.T
    src = pt[0]
    dst = pt[1]

    h = _h_call(atom_features, w1s, b1s)
    nbrh = _gather_k(h, dst)

    bond2 = bond_features.reshape(E // 8, 128)
    nbrh2 = nbrh.reshape(E // 8, 128)
    att8, bf2 = _edge_call(bond2, nbrh2, w2bd, b2rep, s3, b3rep, p)
    att1 = att8.reshape(E)

    zf = jnp.zeros((NPAD,), f32)
    zi = jnp.zeros((NPAD,), i32)
    attn = _norm_k(att1, src, zf, zi)

    z16 = jnp.zeros((NPAD, U), f32)
    partials, out2t = _scat_k(nbrh, bf2.reshape(E, U), attn, src, z16)
    out1 = _comb_call(partials)
    return (out1, out2t.T)
